# baseline (speedup 1.0000x reference)
import jax
import jax.numpy as jnp
from jax.experimental import pallas as pl
from jax.experimental.pallas import tpu as pltpu

_WORDS = 4096


def _combine_body(idx0_ref, idx1_ref, mask_ref, a_ref, b_ref, o_ref):
    i = pl.program_id(0)
    m = mask_ref[i]
    a = a_ref[...]
    b = b_ref[...]
    o_ref[...] = jnp.bitwise_not(jnp.where(m != 0, a | b, a & b))


def _gather_combine(x, idx0, idx1, mask):
    n, w = x.shape
    x3 = x.reshape(n, 1, w)
    out = pl.pallas_call(
        _combine_body,
        grid_spec=pltpu.PrefetchScalarGridSpec(
            num_scalar_prefetch=3,
            grid=(n,),
            in_specs=[
                pl.BlockSpec((1, 1, w), lambda i, i0, i1, m: (i0[i], 0, 0)),
                pl.BlockSpec((1, 1, w), lambda i, i0, i1, m: (i1[i], 0, 0)),
            ],
            out_specs=pl.BlockSpec((1, 1, w), lambda i, i0, i1, m: (i, 0, 0)),
        ),
        out_shape=jax.ShapeDtypeStruct((n, 1, w), jnp.int32),
    )(idx0, idx1, mask, x3, x3)
    return out.reshape(n, w)


def kernel(input_bitarrays, output_shape, adj_logits_0, nor_logits_0, adj_temp_0, nor_temp_0, adj_logits_1, nor_logits_1, adj_temp_1, nor_temp_1, adj_logits_2, nor_logits_2, adj_temp_2, nor_temp_2, adj_logits_3, nor_logits_3, adj_temp_3, nor_temp_3, adj_logits_4, nor_logits_4, adj_temp_4, nor_temp_4):
    params = {
        'adj_logits_0': adj_logits_0, 'nor_logits_0': nor_logits_0,
        'adj_temp_0': adj_temp_0, 'nor_temp_0': nor_temp_0,
        'adj_logits_1': adj_logits_1, 'nor_logits_1': nor_logits_1,
        'adj_temp_1': adj_temp_1, 'nor_temp_1': nor_temp_1,
        'adj_logits_2': adj_logits_2, 'nor_logits_2': nor_logits_2,
        'adj_temp_2': adj_temp_2, 'nor_temp_2': nor_temp_2,
        'adj_logits_3': adj_logits_3, 'nor_logits_3': nor_logits_3,
        'adj_temp_3': adj_temp_3, 'nor_temp_3': nor_temp_3,
        'adj_logits_4': adj_logits_4, 'nor_logits_4': nor_logits_4,
        'adj_temp_4': adj_temp_4, 'nor_temp_4': nor_temp_4,
    }
    key = jax.random.key(42)
    x = jnp.bitwise_or(input_bitarrays, jnp.int32(0) * output_shape)
    for i in range(5):
        al = params[f'adj_logits_{i}']
        at = params[f'adj_temp_{i}']
        nl = params[f'nor_logits_{i}']
        nt = params[f'nor_temp_{i}']
        k = jax.random.fold_in(key, i)
        k1, k2, k3 = jax.random.split(k, 3)
        logits = al * at
        idx0 = jax.random.categorical(k1, logits, axis=-1).astype(jnp.int32)
        idx1 = jax.random.categorical(k2, logits, axis=-1).astype(jnp.int32)
        nor_mask = jax.random.bernoulli(k3, jax.nn.sigmoid(nl * nt))
        x = _gather_combine(x, idx0, idx1, nor_mask.astype(jnp.int32))
    return x


# speedup vs baseline: 3.6734x; 3.6734x over previous
import functools

import jax
import jax.numpy as jnp
from jax import lax
from jax.experimental import pallas as pl
from jax.experimental.pallas import tpu as pltpu
from jax.experimental.pallas import tpu_sc as plsc

_N = 4096
_W = 4096
_NWORK = 32
_PER_W = _N // _NWORK
_C = 4
_NCHUNK = _PER_W // _C
_L = 16


def _layer_body(x_hbm, idx0_hbm, idx1_hbm, mmb_hbm, out_hbm,
                idx0_v, idx1_v, mmb_v, a_v, b_v, o_v, sem_a, sem_b):
    wid = lax.axis_index("s") * 2 + lax.axis_index("c")
    pltpu.sync_copy(idx0_hbm.at[wid], idx0_v)
    pltpu.sync_copy(idx1_hbm.at[wid], idx1_v)
    pltpu.sync_copy(mmb_hbm.at[wid], mmb_v)

    def chunk_body(g, carry):
        ca = pltpu.async_copy(x_hbm.at[idx0_v.at[g]], a_v, sem_a)
        cb = pltpu.async_copy(x_hbm.at[idx1_v.at[g]], b_v, sem_b)
        ca.wait()
        cb.wait()
        for c in range(_C):
            mm = mmb_v[g * _C + c, :]
            nm = jnp.bitwise_not(mm)

            def wbody(w, _, c=c, mm=mm, nm=nm):
                sl = pl.ds(w * _L, _L)
                a = a_v[c, sl]
                b = b_v[c, sl]
                o_v[c, sl] = ((a ^ mm) & (b ^ mm)) ^ nm
                return 0

            lax.fori_loop(0, _W // _L, wbody, 0, unroll=8)
        base = wid * _PER_W + g * _C
        pltpu.sync_copy(o_v, out_hbm.at[pl.ds(base, _C)])
        return carry

    lax.fori_loop(0, _NCHUNK, chunk_body, 0)


_sc_layer = functools.partial(
    pl.kernel,
    mesh=plsc.VectorSubcoreMesh(core_axis_name="c", subcore_axis_name="s"),
    out_type=jax.ShapeDtypeStruct((_N, _W), jnp.int32),
    scratch_types=[
        pltpu.VMEM((_NCHUNK, _C), jnp.int32),
        pltpu.VMEM((_NCHUNK, _C), jnp.int32),
        pltpu.VMEM((_PER_W, _L), jnp.int32),
        pltpu.VMEM((_C, _W), jnp.int32),
        pltpu.VMEM((_C, _W), jnp.int32),
        pltpu.VMEM((_C, _W), jnp.int32),
        pltpu.SemaphoreType.DMA,
        pltpu.SemaphoreType.DMA,
    ],
)(_layer_body)


def _gather_combine(x, idx0, idx1, nor_mask):
    idx0c = idx0.reshape(_NWORK, _NCHUNK, _C)
    idx1c = idx1.reshape(_NWORK, _NCHUNK, _C)
    mm = jnp.where(nor_mask, jnp.int32(-1), jnp.int32(0))
    mmb = jnp.broadcast_to(mm[:, None], (_N, _L)).reshape(_NWORK, _PER_W, _L)
    return _sc_layer(x, idx0c, idx1c, mmb)


def kernel(input_bitarrays, output_shape, adj_logits_0, nor_logits_0, adj_temp_0, nor_temp_0, adj_logits_1, nor_logits_1, adj_temp_1, nor_temp_1, adj_logits_2, nor_logits_2, adj_temp_2, nor_temp_2, adj_logits_3, nor_logits_3, adj_temp_3, nor_temp_3, adj_logits_4, nor_logits_4, adj_temp_4, nor_temp_4):
    params = {
        'adj_logits_0': adj_logits_0, 'nor_logits_0': nor_logits_0,
        'adj_temp_0': adj_temp_0, 'nor_temp_0': nor_temp_0,
        'adj_logits_1': adj_logits_1, 'nor_logits_1': nor_logits_1,
        'adj_temp_1': adj_temp_1, 'nor_temp_1': nor_temp_1,
        'adj_logits_2': adj_logits_2, 'nor_logits_2': nor_logits_2,
        'adj_temp_2': adj_temp_2, 'nor_temp_2': nor_temp_2,
        'adj_logits_3': adj_logits_3, 'nor_logits_3': nor_logits_3,
        'adj_temp_3': adj_temp_3, 'nor_temp_3': nor_temp_3,
        'adj_logits_4': adj_logits_4, 'nor_logits_4': nor_logits_4,
        'adj_temp_4': adj_temp_4, 'nor_temp_4': nor_temp_4,
    }
    key = jax.random.key(42)
    x = jnp.bitwise_or(input_bitarrays, jnp.int32(0) * output_shape)
    for i in range(5):
        al = params[f'adj_logits_{i}']
        at = params[f'adj_temp_{i}']
        nl = params[f'nor_logits_{i}']
        nt = params[f'nor_temp_{i}']
        k = jax.random.fold_in(key, i)
        k1, k2, k3 = jax.random.split(k, 3)
        logits = al * at
        idx0 = jax.random.categorical(k1, logits, axis=-1).astype(jnp.int32)
        idx1 = jax.random.categorical(k2, logits, axis=-1).astype(jnp.int32)
        nor_mask = jax.random.bernoulli(k3, jax.nn.sigmoid(nl * nt))
        x = _gather_combine(x, idx0, idx1, nor_mask)
    return x


# speedup vs baseline: 3.8142x; 1.0383x over previous
import functools

import jax
import jax.numpy as jnp
from jax import lax
from jax.experimental import pallas as pl
from jax.experimental.pallas import tpu as pltpu
from jax.experimental.pallas import tpu_sc as plsc

_N = 4096
_W = 4096
_NWORK = 32
_PER_W = _N // _NWORK
_C = 4
_NCHUNK = _PER_W // _C
_L = 16


def _layer_body(x_hbm, idx0_hbm, idx1_hbm, mmb_hbm, out_hbm,
                idx0_v, idx1_v, mmb_v, a_v, b_v, o_v,
                sem_a0, sem_a1, sem_b0, sem_b1, sem_o0, sem_o1):
    wid = lax.axis_index("s") * 2 + lax.axis_index("c")
    pltpu.sync_copy(idx0_hbm.at[wid], idx0_v)
    pltpu.sync_copy(idx1_hbm.at[wid], idx1_v)
    pltpu.sync_copy(mmb_hbm.at[wid], mmb_v)
    sem_a = (sem_a0, sem_a1)
    sem_b = (sem_b0, sem_b1)
    sem_o = (sem_o0, sem_o1)

    def gather(g, p):
        pltpu.async_copy(x_hbm.at[idx0_v.at[g]], a_v.at[p], sem_a[p])
        pltpu.async_copy(x_hbm.at[idx1_v.at[g]], b_v.at[p], sem_b[p])

    def compute(g, p):
        pltpu.make_async_copy(x_hbm.at[pl.ds(0, _C)], a_v.at[p], sem_a[p]).wait()
        pltpu.make_async_copy(x_hbm.at[pl.ds(0, _C)], b_v.at[p], sem_b[p]).wait()
        for c in range(_C):
            mm = mmb_v[g * _C + c, :]
            nm = jnp.bitwise_not(mm)

            def wbody(w, _, c=c, mm=mm, nm=nm):
                sl = pl.ds(w * _L, _L)
                a = a_v[p, c, sl]
                b = b_v[p, c, sl]
                o_v[p, c, sl] = ((a ^ mm) & (b ^ mm)) ^ nm
                return 0

            lax.fori_loop(0, _W // _L, wbody, 0, unroll=8)
        base = wid * _PER_W + g * _C
        pltpu.async_copy(o_v.at[p], out_hbm.at[pl.ds(base, _C)], sem_o[p])

    gather(0, 0)

    def chunk_body(h, carry):
        g0 = h * 2
        for q in range(2):
            g = g0 + q
            p = q
            nxt = g + 1
            if q == 0:
                gather(nxt, 1)
            else:
                @pl.when(nxt < _NCHUNK)
                def _():
                    gather(nxt, 0)

            @pl.when(g >= 2)
            def _():
                pltpu.make_async_copy(
                    o_v.at[p], out_hbm.at[pl.ds(0, _C)], sem_o[p]).wait()

            compute(g, p)
        return carry

    lax.fori_loop(0, _NCHUNK // 2, chunk_body, 0)
    pltpu.make_async_copy(o_v.at[0], out_hbm.at[pl.ds(0, _C)], sem_o[0]).wait()
    pltpu.make_async_copy(o_v.at[1], out_hbm.at[pl.ds(0, _C)], sem_o[1]).wait()


_sc_layer = functools.partial(
    pl.kernel,
    mesh=plsc.VectorSubcoreMesh(core_axis_name="c", subcore_axis_name="s"),
    out_type=jax.ShapeDtypeStruct((_N, _W), jnp.int32),
    scratch_types=[
        pltpu.VMEM((_NCHUNK, _C), jnp.int32),
        pltpu.VMEM((_NCHUNK, _C), jnp.int32),
        pltpu.VMEM((_PER_W, _L), jnp.int32),
        pltpu.VMEM((2, _C, _W), jnp.int32),
        pltpu.VMEM((2, _C, _W), jnp.int32),
        pltpu.VMEM((2, _C, _W), jnp.int32),
        pltpu.SemaphoreType.DMA,
        pltpu.SemaphoreType.DMA,
        pltpu.SemaphoreType.DMA,
        pltpu.SemaphoreType.DMA,
        pltpu.SemaphoreType.DMA,
        pltpu.SemaphoreType.DMA,
    ],
)(_layer_body)


def _gather_combine(x, idx0, idx1, nor_mask):
    idx0c = idx0.reshape(_NWORK, _NCHUNK, _C)
    idx1c = idx1.reshape(_NWORK, _NCHUNK, _C)
    mm = jnp.where(nor_mask, jnp.int32(-1), jnp.int32(0))
    mmb = jnp.broadcast_to(mm[:, None], (_N, _L)).reshape(_NWORK, _PER_W, _L)
    return _sc_layer(x, idx0c, idx1c, mmb)


def kernel(input_bitarrays, output_shape, adj_logits_0, nor_logits_0, adj_temp_0, nor_temp_0, adj_logits_1, nor_logits_1, adj_temp_1, nor_temp_1, adj_logits_2, nor_logits_2, adj_temp_2, nor_temp_2, adj_logits_3, nor_logits_3, adj_temp_3, nor_temp_3, adj_logits_4, nor_logits_4, adj_temp_4, nor_temp_4):
    params = {
        'adj_logits_0': adj_logits_0, 'nor_logits_0': nor_logits_0,
        'adj_temp_0': adj_temp_0, 'nor_temp_0': nor_temp_0,
        'adj_logits_1': adj_logits_1, 'nor_logits_1': nor_logits_1,
        'adj_temp_1': adj_temp_1, 'nor_temp_1': nor_temp_1,
        'adj_logits_2': adj_logits_2, 'nor_logits_2': nor_logits_2,
        'adj_temp_2': adj_temp_2, 'nor_temp_2': nor_temp_2,
        'adj_logits_3': adj_logits_3, 'nor_logits_3': nor_logits_3,
        'adj_temp_3': adj_temp_3, 'nor_temp_3': nor_temp_3,
        'adj_logits_4': adj_logits_4, 'nor_logits_4': nor_logits_4,
        'adj_temp_4': adj_temp_4, 'nor_temp_4': nor_temp_4,
    }
    key = jax.random.key(42)
    x = jnp.bitwise_or(input_bitarrays, jnp.int32(0) * output_shape)
    for i in range(5):
        al = params[f'adj_logits_{i}']
        at = params[f'adj_temp_{i}']
        nl = params[f'nor_logits_{i}']
        nt = params[f'nor_temp_{i}']
        k = jax.random.fold_in(key, i)
        k1, k2, k3 = jax.random.split(k, 3)
        logits = al * at
        idx0 = jax.random.categorical(k1, logits, axis=-1).astype(jnp.int32)
        idx1 = jax.random.categorical(k2, logits, axis=-1).astype(jnp.int32)
        nor_mask = jax.random.bernoulli(k3, jax.nn.sigmoid(nl * nt))
        x = _gather_combine(x, idx0, idx1, nor_mask)
    return x
